# baseline (speedup 1.0000x reference)
import jax
import jax.numpy as jnp
from jax import lax
from jax.experimental import pallas as pl
from jax.experimental.pallas import tpu as pltpu
from jax.experimental.pallas import tpu_sc as plsc

_BATCH = 16384
_D = 64
_LANES = 16
_NC = 2
_NS = 16
_NW = _NC * _NS
_BPW = _BATCH // _NW
_CHUNK = 128
_NCHUNK = _BPW // _CHUNK
_GROUPS_PER_CHUNK = _CHUNK // _LANES
_Y_HIGH = 5.5


def _body(x_hbm, uf_hbm, ub_hbm, mf_hbm, mb_hbm, out_hbm,
          xv, uidx, midx, urows, mrows, ubias, mbias, res, sem):
    wid = lax.axis_index("s") * _NC + lax.axis_index("c")
    base = wid * _BPW
    pltpu.sync_copy(x_hbm.at[pl.ds(base * 2, _BPW * 2)], xv)

    lane = lax.iota(jnp.int32, _LANES)
    for j in range(_NCHUNK):
        for g in range(_GROUPS_PER_CHUNK):
            flat = lane * 2 + (j * _CHUNK + g * _LANES) * 2
            uidx[j, pl.ds(g * _LANES, _LANES)] = plsc.load_gather(xv, [flat])
            midx[j, pl.ds(g * _LANES, _LANES)] = plsc.load_gather(xv, [flat + 1])

    copies = []
    for j in range(_NCHUNK):
        rsl = pl.ds(j * _CHUNK, _CHUNK)
        copies.append(pltpu.async_copy(uf_hbm.at[uidx.at[j]], urows.at[rsl], sem))
        copies.append(pltpu.async_copy(mf_hbm.at[midx.at[j]], mrows.at[rsl], sem))
        copies.append(pltpu.async_copy(ub_hbm.at[uidx.at[j]], ubias.at[rsl], sem))
        copies.append(pltpu.async_copy(mb_hbm.at[midx.at[j]], mbias.at[rsl], sem))
    for c in copies:
        c.wait()

    def group_body(g, carry):
        acc = jnp.zeros((_LANES,), jnp.float32)
        for rr in range(_LANES):
            r = g * _LANES + rr
            s = urows[r, pl.ds(0, _LANES)] * mrows[r, pl.ds(0, _LANES)]
            for k in range(1, _D // _LANES):
                s = s + (urows[r, pl.ds(k * _LANES, _LANES)] *
                         mrows[r, pl.ds(k * _LANES, _LANES)])
            acc = jnp.where(lane == rr, jnp.sum(s), acc)
        sl = pl.ds(g * _LANES, _LANES)
        acc = acc + ubias[sl] + mbias[sl]
        res[sl] = _Y_HIGH / (1.0 + jnp.exp(-acc))
        return carry

    lax.fori_loop(0, _BPW // _LANES, group_body, 0)
    pltpu.sync_copy(res, out_hbm.at[pl.ds(base, _BPW)])


@jax.jit
def kernel(x, user_factors, user_bias, movie_factors, movie_bias):
    f = pl.kernel(
        _body,
        out_type=jax.ShapeDtypeStruct((_BATCH,), jnp.float32),
        mesh=plsc.VectorSubcoreMesh(core_axis_name="c", subcore_axis_name="s"),
        compiler_params=pltpu.CompilerParams(
            needs_layout_passes=False, use_tc_tiling_on_sc=False),
        scratch_types=[
            pltpu.VMEM((_BPW * 2,), jnp.int32),
            pltpu.VMEM((_NCHUNK, _CHUNK), jnp.int32),
            pltpu.VMEM((_NCHUNK, _CHUNK), jnp.int32),
            pltpu.VMEM((_BPW, _D), jnp.float32),
            pltpu.VMEM((_BPW, _D), jnp.float32),
            pltpu.VMEM((_BPW,), jnp.float32),
            pltpu.VMEM((_BPW,), jnp.float32),
            pltpu.VMEM((_BPW,), jnp.float32),
            pltpu.SemaphoreType.DMA,
        ],
    )
    out = f(x.reshape(_BATCH * 2), user_factors,
            user_bias.reshape(user_bias.shape[0]), movie_factors,
            movie_bias.reshape(movie_bias.shape[0]))
    return out.reshape(_BATCH, 1)
